# Initial kernel scaffold; baseline (speedup 1.0000x reference)
import jax
import jax.numpy as jnp
from jax.experimental import pallas as pl

_R = 256


def _suffix_max_argmax(vals):
    n = vals.shape[-1]
    bb = vals.shape[0]
    val = vals
    idx = jax.lax.broadcasted_iota(jnp.int32, vals.shape, 1)
    k = 1
    while k < n:
        pad_v = jnp.full((bb, k), -jnp.inf, jnp.float32)
        pad_i = jnp.full((bb, k), 2 * n, jnp.int32)
        val_s = jnp.concatenate([val[:, k:], pad_v], axis=1)
        idx_s = jnp.concatenate([idx[:, k:], pad_i], axis=1)
        take = val_s > val
        val = jnp.where(take, val_s, val)
        idx = jnp.where(take, idx_s, idx)
        k *= 2
    return val, idx


def _fused_body(x_ref, wt_ref, o_ref):
    bb = x_ref.shape[0] // _R
    acc = jnp.dot(x_ref[...], wt_ref[...], preferred_element_type=jnp.float32)
    mo = jnp.where(acc == 0.0, -1.0, acc).reshape(bb, _R, _R)

    riota = jax.lax.broadcasted_iota(jnp.int32, (bb, _R, _R), 1)
    oiota = jax.lax.broadcasted_iota(jnp.int32, (bb, _R, _R), 2)

    colmax = jnp.max(mo, axis=1)
    mc = jnp.min(jnp.where(mo == colmax[:, None, :], riota, 2 * _R), axis=1)
    rowmax = jnp.max(mo, axis=2)
    ra = jnp.min(jnp.where(mo == rowmax[:, :, None], oiota, 2 * _R), axis=2)
    smr = jnp.sum(jnp.where(riota == mc[:, None, :], ra[:, :, None], 0), axis=1)

    L = mo[:, _R - 1, :]
    SM, SA = _suffix_max_argmax(L)

    liota = jax.lax.broadcasted_iota(jnp.int32, (bb, _R), 1)
    a = SA == liota
    stat = smr != liota
    is_dyn = mc == (_R - 1)
    thresh = jnp.where(liota == 0, -jnp.inf, 0.0)
    pre_dyn = jnp.logical_not(jnp.logical_and(a, SM > thresh))
    post_dyn = jnp.logical_not(jnp.logical_and(a, SM > 1.0))
    out_pre = jnp.where(is_dyn, pre_dyn, stat)
    out_post = jnp.where(is_dyn, post_dyn, stat)
    t = jnp.min(
        jnp.where(jnp.logical_and(out_pre, liota < _R - 1), liota, 2 * _R),
        axis=1, keepdims=True)
    out_f = jnp.where(liota <= t, out_pre, out_post).astype(jnp.float32)

    fin = jnp.where(mo == -1.0, 0.0, mo)
    lastfin = fin[:, _R - 1, :]
    newlast = jnp.where(liota < _R - 1, out_f, lastfin)
    fin = jnp.where(riota == _R - 1, newlast[:, None, :], fin)
    o_ref[...] = fin.reshape(bb * _R, _R)


def _run(x, W, bb, interpret=False):
    B, R, D = x.shape
    O = W.shape[0]
    xr = x.reshape(B * R, D)
    wt = W.T
    fin = pl.pallas_call(
        _fused_body,
        grid=(B // bb,),
        in_specs=[
            pl.BlockSpec((bb * R, D), lambda i: (i, 0)),
            pl.BlockSpec((D, O), lambda i: (0, 0)),
        ],
        out_specs=pl.BlockSpec((bb * R, O), lambda i: (i, 0)),
        out_shape=jax.ShapeDtypeStruct((B * R, O), jnp.float32),
        interpret=interpret,
    )(xr, wt)
    fin = fin.reshape(B, R, O)
    t2 = jnp.transpose(fin[:, :, : O - 1], (2, 0, 1))
    return fin, t2


def kernel(x, W):
    return _run(x, W, bb=2)


# speedup vs baseline: 132.1203x; 132.1203x over previous
import jax
import jax.numpy as jnp
from jax.experimental import pallas as pl

_R = 256


def _suffix_max_argmax(vals):
    n = vals.shape[-1]
    bb = vals.shape[0]
    val = vals
    idx = jax.lax.broadcasted_iota(jnp.int32, vals.shape, 1)
    k = 1
    while k < n:
        pad_v = jnp.full((bb, k), -jnp.inf, jnp.float32)
        pad_i = jnp.full((bb, k), 2 * n, jnp.int32)
        val_s = jnp.concatenate([val[:, k:], pad_v], axis=1)
        idx_s = jnp.concatenate([idx[:, k:], pad_i], axis=1)
        take = val_s > val
        val = jnp.where(take, val_s, val)
        idx = jnp.where(take, idx_s, idx)
        k *= 2
    return val, idx


def _fused_body(x_ref, wt_ref, o_ref):
    bb = x_ref.shape[0] // _R
    acc = jnp.dot(x_ref[...], wt_ref[...], preferred_element_type=jnp.float32)
    mo = jnp.where(acc == 0.0, -1.0, acc).reshape(bb, _R, _R)

    riota = jax.lax.broadcasted_iota(jnp.int32, (bb, _R, _R), 1)
    oiota = jax.lax.broadcasted_iota(jnp.int32, (bb, _R, _R), 2)

    colmax = jnp.max(mo, axis=1)
    mc = jnp.min(jnp.where(mo == colmax[:, None, :], riota, 2 * _R), axis=1)
    rowmax = jnp.max(mo, axis=2)
    ra = jnp.min(jnp.where(mo == rowmax[:, :, None], oiota, 2 * _R), axis=2)
    smr = jnp.sum(jnp.where(riota == mc[:, None, :], ra[:, :, None], 0), axis=1)

    L = jnp.max(jnp.where(riota == _R - 1, mo, -jnp.inf), axis=1)
    SM, SA = _suffix_max_argmax(L)

    liota = jax.lax.broadcasted_iota(jnp.int32, (bb, _R), 1)
    a_f = jnp.where(SA == liota, 1.0, 0.0)
    stat_f = jnp.where(smr != liota, 1.0, 0.0)
    thresh = jnp.where(liota == 0, -jnp.inf, 0.0)
    gt_pre = jnp.where(SM > thresh, 1.0, 0.0)
    gt_post = jnp.where(SM > 1.0, 1.0, 0.0)
    pre_dyn = 1.0 - a_f * gt_pre
    post_dyn = 1.0 - a_f * gt_post
    out_pre = jnp.where(mc == _R - 1, pre_dyn, stat_f)
    out_post = jnp.where(mc == _R - 1, post_dyn, stat_f)
    t = jnp.min(
        jnp.where(out_pre > 0.5,
                  jnp.where(liota < _R - 1, liota, 2 * _R), 2 * _R),
        axis=1, keepdims=True)
    out_f = jnp.where(liota <= t, out_pre, out_post)

    fin = jnp.where(mo == -1.0, 0.0, mo)
    lastfin = jnp.where(L == -1.0, 0.0, L)
    newlast = jnp.where(liota < _R - 1, out_f, lastfin)
    fin = jnp.where(riota == _R - 1, newlast[:, None, :], fin)
    o_ref[...] = fin.reshape(bb * _R, _R)


def _run(x, W, bb, interpret=False):
    B, R, D = x.shape
    O = W.shape[0]
    xr = x.reshape(B * R, D)
    wt = W.T
    fin = pl.pallas_call(
        _fused_body,
        grid=(B // bb,),
        in_specs=[
            pl.BlockSpec((bb * R, D), lambda i: (i, 0)),
            pl.BlockSpec((D, O), lambda i: (0, 0)),
        ],
        out_specs=pl.BlockSpec((bb * R, O), lambda i: (i, 0)),
        out_shape=jax.ShapeDtypeStruct((B * R, O), jnp.float32),
        interpret=interpret,
    )(xr, wt)
    fin = fin.reshape(B, R, O)
    t2 = jnp.transpose(fin[:, :, : O - 1], (2, 0, 1))
    return fin, t2


def kernel(x, W):
    return _run(x, W, bb=2)


# speedup vs baseline: 227.7304x; 1.7237x over previous
import jax
import jax.numpy as jnp
from jax.experimental import pallas as pl

_R = 256


def _suffix_max_argmax(vals):
    n = vals.shape[-1]
    bb = vals.shape[0]
    val = vals
    idx = jax.lax.broadcasted_iota(jnp.int32, vals.shape, 1)
    k = 1
    while k < n:
        pad_v = jnp.full((bb, k), -jnp.inf, jnp.float32)
        pad_i = jnp.full((bb, k), 2 * n, jnp.int32)
        val_s = jnp.concatenate([val[:, k:], pad_v], axis=1)
        idx_s = jnp.concatenate([idx[:, k:], pad_i], axis=1)
        take = val_s > val
        val = jnp.where(take, val_s, val)
        idx = jnp.where(take, idx_s, idx)
        k *= 2
    return val, idx


def _fused_body(x_ref, wt_ref, o_ref, t2_ref):
    bb = x_ref.shape[0] // _R
    acc = jnp.dot(x_ref[...], wt_ref[...], preferred_element_type=jnp.float32)
    mo = jnp.where(acc == 0.0, -1.0, acc).reshape(bb, _R, _R)

    riota = jax.lax.broadcasted_iota(jnp.int32, (bb, _R, _R), 1)
    oiota = jax.lax.broadcasted_iota(jnp.int32, (bb, _R, _R), 2)

    colmax = jnp.max(mo, axis=1)
    mc = jnp.min(jnp.where(mo == colmax[:, None, :], riota, 2 * _R), axis=1)
    rowmax = jnp.max(mo, axis=2)
    ra = jnp.min(jnp.where(mo == rowmax[:, :, None], oiota, 2 * _R), axis=2)
    smr = jnp.sum(jnp.where(riota == mc[:, None, :], ra[:, :, None], 0), axis=1)

    L = jnp.max(jnp.where(riota == _R - 1, mo, -jnp.inf), axis=1)
    SM, SA = _suffix_max_argmax(L)

    liota = jax.lax.broadcasted_iota(jnp.int32, (bb, _R), 1)
    a_f = jnp.where(SA == liota, 1.0, 0.0)
    stat_f = jnp.where(smr != liota, 1.0, 0.0)
    thresh = jnp.where(liota == 0, -jnp.inf, 0.0)
    gt_pre = jnp.where(SM > thresh, 1.0, 0.0)
    gt_post = jnp.where(SM > 1.0, 1.0, 0.0)
    pre_dyn = 1.0 - a_f * gt_pre
    post_dyn = 1.0 - a_f * gt_post
    out_pre = jnp.where(mc == _R - 1, pre_dyn, stat_f)
    out_post = jnp.where(mc == _R - 1, post_dyn, stat_f)
    t = jnp.min(
        jnp.where(out_pre > 0.5,
                  jnp.where(liota < _R - 1, liota, 2 * _R), 2 * _R),
        axis=1, keepdims=True)
    out_f = jnp.where(liota <= t, out_pre, out_post)

    fin = jnp.where(mo == -1.0, 0.0, mo)
    lastfin = jnp.where(L == -1.0, 0.0, L)
    newlast = jnp.where(liota < _R - 1, out_f, lastfin)
    fin = jnp.where(riota == _R - 1, newlast[:, None, :], fin)
    o_ref[...] = fin.reshape(bb * _R, _R)
    for b in range(bb):
        t2_ref[:, b, :] = jnp.transpose(fin[b, :, :])


def _run(x, W, bb, interpret=False):
    B, R, D = x.shape
    O = W.shape[0]
    xr = x.reshape(B * R, D)
    wt = W.T
    fin, t2 = pl.pallas_call(
        _fused_body,
        grid=(B // bb,),
        in_specs=[
            pl.BlockSpec((bb * R, D), lambda i: (i, 0)),
            pl.BlockSpec((D, O), lambda i: (0, 0)),
        ],
        out_specs=[
            pl.BlockSpec((bb * R, O), lambda i: (i, 0)),
            pl.BlockSpec((O, bb, R), lambda i: (0, i, 0)),
        ],
        out_shape=[
            jax.ShapeDtypeStruct((B * R, O), jnp.float32),
            jax.ShapeDtypeStruct((O - 1, B, R), jnp.float32),
        ],
        interpret=interpret,
    )(xr, wt)
    return fin.reshape(B, R, O), t2


def kernel(x, W):
    return _run(x, W, bb=8)


# speedup vs baseline: 228.1643x; 1.0019x over previous
import jax
import jax.numpy as jnp
from jax.experimental import pallas as pl

_R = 256


def _suffix_max_argmax(vals):
    n = vals.shape[-1]
    bb = vals.shape[0]
    val = vals
    idx = jax.lax.broadcasted_iota(jnp.int32, vals.shape, 1)
    k = 1
    while k < n:
        pad_v = jnp.full((bb, k), -jnp.inf, jnp.float32)
        pad_i = jnp.full((bb, k), 2 * n, jnp.int32)
        val_s = jnp.concatenate([val[:, k:], pad_v], axis=1)
        idx_s = jnp.concatenate([idx[:, k:], pad_i], axis=1)
        take = val_s > val
        val = jnp.where(take, val_s, val)
        idx = jnp.where(take, idx_s, idx)
        k *= 2
    return val, idx


def _fused_body(x_ref, wt_ref, o_ref, t2_ref):
    bb = x_ref.shape[0] // _R
    part = jnp.dot(x_ref[...], wt_ref[...], preferred_element_type=jnp.float32)
    _epilogue(bb, part, o_ref, t2_ref)


def _epilogue(bb, acc, o_ref, t2_ref):
    mo = jnp.where(acc == 0.0, -1.0, acc).reshape(bb, _R, _R)

    riota = jax.lax.broadcasted_iota(jnp.int32, (bb, _R, _R), 1)
    oiota = jax.lax.broadcasted_iota(jnp.int32, (bb, _R, _R), 2)

    colmax = jnp.max(mo, axis=1)
    mc = jnp.min(jnp.where(mo == colmax[:, None, :], riota, 2 * _R), axis=1)
    rowmax = jnp.max(mo, axis=2)
    ra = jnp.min(jnp.where(mo == rowmax[:, :, None], oiota, 2 * _R), axis=2)
    onehot = jnp.where(riota == mc[:, None, :], 1.0, 0.0)
    smr = jax.lax.dot_general(
        ra.astype(jnp.float32), onehot,
        (((1,), (1,)), ((0,), (0,))),
        preferred_element_type=jnp.float32)

    L = jnp.max(jnp.where(riota == _R - 1, mo, -jnp.inf), axis=1)
    SM, SA = _suffix_max_argmax(L)

    liota = jax.lax.broadcasted_iota(jnp.int32, (bb, _R), 1)
    a_f = jnp.where(SA == liota, 1.0, 0.0)
    stat_f = jnp.where(smr != liota, 1.0, 0.0)
    thresh = jnp.where(liota == 0, -jnp.inf, 0.0)
    gt_pre = jnp.where(SM > thresh, 1.0, 0.0)
    gt_post = jnp.where(SM > 1.0, 1.0, 0.0)
    pre_dyn = 1.0 - a_f * gt_pre
    post_dyn = 1.0 - a_f * gt_post
    out_pre = jnp.where(mc == _R - 1, pre_dyn, stat_f)
    out_post = jnp.where(mc == _R - 1, post_dyn, stat_f)
    t = jnp.min(
        jnp.where(out_pre > 0.5,
                  jnp.where(liota < _R - 1, liota, 2 * _R), 2 * _R),
        axis=1, keepdims=True)
    out_f = jnp.where(liota <= t, out_pre, out_post)

    fin = jnp.where(mo == -1.0, 0.0, mo)
    lastfin = jnp.where(L == -1.0, 0.0, L)
    newlast = jnp.where(liota < _R - 1, out_f, lastfin)
    fin = jnp.where(riota == _R - 1, newlast[:, None, :], fin)
    o_ref[...] = fin.reshape(bb * _R, _R)
    for b in range(bb):
        t2_ref[:, b, :] = jnp.transpose(fin[b, :, :])


def _run(x, W, bb, interpret=False):
    B, R, D = x.shape
    O = W.shape[0]
    xr = x.reshape(B * R, D)
    wt = W.T
    fin, t2 = pl.pallas_call(
        _fused_body,
        grid=(B // bb,),
        in_specs=[
            pl.BlockSpec((bb * R, D), lambda i: (i, 0)),
            pl.BlockSpec((D, O), lambda i: (0, 0)),
        ],
        out_specs=[
            pl.BlockSpec((bb * R, O), lambda i: (i, 0)),
            pl.BlockSpec((O, bb, R), lambda i: (0, i, 0)),
        ],
        out_shape=[
            jax.ShapeDtypeStruct((B * R, O), jnp.float32),
            jax.ShapeDtypeStruct((O - 1, B, R), jnp.float32),
        ],
        interpret=interpret,
    )(xr, wt)
    return fin.reshape(B, R, O), t2


def kernel(x, W):
    return _run(x, W, bb=8)
